# baseline (speedup 1.0000x reference)
import jax
import jax.numpy as jnp
from jax import lax
from jax.experimental import pallas as pl
from jax.experimental.pallas import tpu as pltpu
from jax.experimental.pallas import tpu_sc as plsc

EMBED = 100
SEQ = 200
LABEL = 1000

NC = 2
NS = 16

NEG = -3.0e38


def _row_max(rows_ref, k, lane, acc):
    for off in (0, 16, 32, 48, 64, 80):
        acc = jnp.maximum(acc, rows_ref[k, 0, pl.ds(off, 16)])
    tail = plsc.load_gather(
        rows_ref,
        [jnp.full((16,), k, jnp.int32), jnp.zeros((16,), jnp.int32),
         lane + 84])
    return jnp.maximum(acc, tail)


def _pool_rows(idx_hbm, table_hbm, pooled_hbm, idxv, rows, pv, sem,
               idx_off, n_rows, out_off):
    pltpu.sync_copy(idx_hbm.at[pl.ds(idx_off, n_rows)],
                    idxv.at[pl.ds(0, n_rows)])
    iv = idxv[...]
    lane = lax.iota(jnp.int32, 16)
    copies = []
    for k in range(n_rows):
        copies.append(
            pltpu.async_copy(table_hbm.at[pl.ds(iv[k], 1)],
                             rows.at[k], sem))
    for c in copies:
        c.wait()
    out = jnp.zeros((16,), jnp.float32)
    for p in range(n_rows // 2):
        acc = jnp.full((16,), NEG, jnp.float32)
        acc = _row_max(rows, 2 * p, lane, acc)
        acc = _row_max(rows, 2 * p + 1, lane, acc)
        m = jnp.max(acc)
        out = jnp.where(lane == p, jnp.full((16,), m, jnp.float32), out)
    pv[...] = out
    pltpu.sync_copy(pv.at[pl.ds(0, 8)], pooled_hbm.at[pl.ds(out_off, 8)])


def _sc_pool_body(idx_hbm, table_hbm, pooled_hbm, idxv, rows, pv, sem):
    c = lax.axis_index("c")
    s = lax.axis_index("s")
    wid = s * NC + c

    @pl.when(wid < 12)
    def _():
        _pool_rows(idx_hbm, table_hbm, pooled_hbm, idxv, rows, pv, sem,
                   wid * 16, 16, wid * 8)

    @pl.when(wid == 12)
    def _():
        _pool_rows(idx_hbm, table_hbm, pooled_hbm, idxv, rows, pv, sem,
                   192, 8, 96)

    @pl.when((wid >= 13) & (wid < 16))
    def _():
        pv[...] = jnp.zeros((16,), jnp.float32)
        pltpu.sync_copy(pv.at[pl.ds(0, 8)], pooled_hbm.at[pl.ds(wid * 8, 8)])


_sc_pool = pl.kernel(
    _sc_pool_body,
    out_type=jax.ShapeDtypeStruct((128,), jnp.float32),
    mesh=plsc.VectorSubcoreMesh(core_axis_name="c", subcore_axis_name="s"),
    compiler_params=pltpu.CompilerParams(
        needs_layout_passes=False, use_tc_tiling_on_sc=True),
    scratch_types=[
        pltpu.VMEM((16,), jnp.int32),
        pltpu.VMEM((16, 1, EMBED), jnp.float32),
        pltpu.VMEM((16,), jnp.float32),
        pltpu.SemaphoreType.DMA,
    ],
)


def _tc_linear_body(pooled_ref, w_ref, b_ref, out_ref):
    p = pooled_ref[:, :EMBED]
    y = lax.dot_general(p, w_ref[...], (((1,), (1,)), ((), ())),
                        preferred_element_type=jnp.float32)
    out_ref[...] = y + b_ref[...]


def kernel(input, emb_table, W, b):
    idx = input.reshape(SEQ).astype(jnp.int32)
    pooled = _sc_pool(idx, emb_table)
    return pl.pallas_call(
        _tc_linear_body,
        out_shape=jax.ShapeDtypeStruct((1, LABEL), jnp.float32),
    )(pooled.reshape(1, 128), W, b.reshape(1, LABEL))


# speedup vs baseline: 11.2465x; 11.2465x over previous
import jax
import jax.numpy as jnp
from jax import lax
from jax.experimental import pallas as pl
from jax.experimental.pallas import tpu as pltpu
from jax.experimental.pallas import tpu_sc as plsc

EMBED = 100
SEQ = 200
LABEL = 1000

NC = 2
NS = 16
NT = 13

NEG = -3.0e38


def _pool_tokens(idx_hbm, table_hbm, pooled_hbm, idxv, tiles, pv, sem,
                 idx_off, n_tok, out_off):
    pltpu.sync_copy(idx_hbm.at[pl.ds(idx_off, n_tok)],
                    idxv.at[pl.ds(0, n_tok)])
    iv = idxv[...]
    lane = lax.iota(jnp.int32, 16)
    lane8 = jnp.bitwise_and(lane, 7)
    out = jnp.zeros((16,), jnp.float32)
    for b0 in range(0, n_tok, 8):
        nb = min(8, n_tok - b0)
        copies = []
        for k in range(nb):
            v = iv[b0 + k]
            cs = pl.multiple_of(
                lax.shift_left(lax.shift_right_logical(v, 7), 7), 128)
            for t in range(NT):
                start = pl.multiple_of(
                    lax.mul(jnp.int32(t), jnp.int32(8)), 8)
                copies.append(
                    pltpu.async_copy(
                        table_hbm.at[pl.ds(start, 8), pl.ds(cs, 128)],
                        tiles.at[k, t], sem))
        for cpy in copies:
            cpy.wait()
        accs = []
        for k in range(nb):
            c = jnp.bitwise_and(iv[b0 + k], 127)
            cvec = jnp.full((16,), c, jnp.int32)
            acc = jnp.full((16,), NEG, jnp.float32)
            for t in range(NT):
                g = plsc.load_gather(
                    tiles,
                    [jnp.full((16,), k, jnp.int32),
                     jnp.full((16,), t, jnp.int32), lane8, cvec])
                if t == NT - 1:
                    g = jnp.where(lane8 < 4, g, NEG)
                acc = jnp.maximum(acc, g)
            accs.append(acc)
        for p in range(nb // 2):
            m = jnp.max(jnp.maximum(accs[2 * p], accs[2 * p + 1]))
            out = jnp.where(lane == b0 // 2 + p,
                            jnp.full((16,), m, jnp.float32), out)
    pv[...] = out
    pltpu.sync_copy(pv.at[pl.ds(0, 8)], pooled_hbm.at[pl.ds(out_off, 8)])


def _sc_pool_body(idx_hbm, table_hbm, pooled_hbm, idxv, tiles, pv, sem):
    c = lax.axis_index("c")
    s = lax.axis_index("s")
    wid = s * NC + c

    @pl.when(wid < 12)
    def _():
        _pool_tokens(idx_hbm, table_hbm, pooled_hbm, idxv, tiles, pv, sem,
                     wid * 16, 16, wid * 8)

    @pl.when(wid == 12)
    def _():
        _pool_tokens(idx_hbm, table_hbm, pooled_hbm, idxv, tiles, pv, sem,
                     192, 8, 96)

    @pl.when((wid >= 13) & (wid < 16))
    def _():
        pv[...] = jnp.zeros((16,), jnp.float32)
        pltpu.sync_copy(pv.at[pl.ds(0, 8)], pooled_hbm.at[pl.ds(wid * 8, 8)])


_sc_pool = pl.kernel(
    _sc_pool_body,
    out_type=jax.ShapeDtypeStruct((128,), jnp.float32),
    mesh=plsc.VectorSubcoreMesh(core_axis_name="c", subcore_axis_name="s"),
    compiler_params=pltpu.CompilerParams(
        needs_layout_passes=False, use_tc_tiling_on_sc=True),
    scratch_types=[
        pltpu.VMEM((16,), jnp.int32),
        pltpu.VMEM((8, NT, 8, 128), jnp.float32),
        pltpu.VMEM((16,), jnp.float32),
        pltpu.SemaphoreType.DMA,
    ],
)


def _tc_linear_body(pooled_ref, wt_ref, b_ref, out_ref):
    p = pooled_ref[:, :EMBED]
    y = lax.dot_general(p, wt_ref[...], (((1,), (0,)), ((), ())),
                        preferred_element_type=jnp.float32)
    out_ref[...] = y + b_ref[...]


def kernel(input, emb_table, W, b):
    idx = input.reshape(SEQ).astype(jnp.int32)
    embT = emb_table.T
    pooled = _sc_pool(idx, embT)
    wt = W.T
    return pl.pallas_call(
        _tc_linear_body,
        out_shape=jax.ShapeDtypeStruct((1, LABEL), jnp.float32),
    )(pooled.reshape(1, 128), wt, b.reshape(1, LABEL))


# speedup vs baseline: 11.2876x; 1.0037x over previous
import jax
import jax.numpy as jnp
from jax import lax
from jax.experimental import pallas as pl
from jax.experimental.pallas import tpu as pltpu
from jax.experimental.pallas import tpu_sc as plsc

EMBED = 100
SEQ = 200
LABEL = 1000

NC = 2
NS = 16
NT = 13

NEG = -3.0e38


NB = 4


def _pool_tokens(idx_hbm, table_hbm, pooled_hbm, idxv, tiles, pv, sems,
                 idx_off, n_tok, out_off):
    pltpu.sync_copy(idx_hbm.at[pl.ds(idx_off, n_tok)],
                    idxv.at[pl.ds(0, n_tok)])
    iv = idxv[...]
    lane = lax.iota(jnp.int32, 16)
    lane8 = jnp.bitwise_and(lane, 7)
    nbatch = n_tok // NB

    def issue(bi):
        buf = bi % 2
        copies = []
        for k in range(NB):
            v = iv[bi * NB + k]
            cs = pl.multiple_of(
                lax.shift_left(lax.shift_right_logical(v, 7), 7), 128)
            for t in range(NT):
                start = pl.multiple_of(
                    lax.mul(jnp.int32(t), jnp.int32(8)), 8)
                copies.append(
                    pltpu.async_copy(
                        table_hbm.at[pl.ds(start, 8), pl.ds(cs, 128)],
                        tiles.at[buf, k, t], sems[buf]))
        return copies

    out = jnp.zeros((16,), jnp.float32)
    inflight = issue(0)
    for bi in range(nbatch):
        for cpy in inflight:
            cpy.wait()
        inflight = issue(bi + 1) if bi + 1 < nbatch else []
        buf = bi % 2
        accs = []
        for k in range(NB):
            c = jnp.bitwise_and(iv[bi * NB + k], 127)
            cvec = jnp.full((16,), c, jnp.int32)
            acc = jnp.full((16,), NEG, jnp.float32)
            for t in range(NT):
                g = plsc.load_gather(
                    tiles,
                    [jnp.full((16,), buf, jnp.int32),
                     jnp.full((16,), k, jnp.int32),
                     jnp.full((16,), t, jnp.int32), lane8, cvec])
                if t == NT - 1:
                    g = jnp.where(lane8 < 4, g, NEG)
                acc = jnp.maximum(acc, g)
            accs.append(acc)
        for p in range(NB // 2):
            m = jnp.max(jnp.maximum(accs[2 * p], accs[2 * p + 1]))
            out = jnp.where(lane == bi * (NB // 2) + p,
                            jnp.full((16,), m, jnp.float32), out)
    pv[...] = out
    pltpu.sync_copy(pv.at[pl.ds(0, 8)], pooled_hbm.at[pl.ds(out_off, 8)])


def _sc_pool_body(idx_hbm, table_hbm, pooled_hbm, idxv, tiles, pv, sem0, sem1):
    c = lax.axis_index("c")
    s = lax.axis_index("s")
    wid = s * NC + c

    @pl.when(wid < 12)
    def _():
        _pool_tokens(idx_hbm, table_hbm, pooled_hbm, idxv, tiles, pv,
                     (sem0, sem1), wid * 16, 16, wid * 8)

    @pl.when(wid == 12)
    def _():
        _pool_tokens(idx_hbm, table_hbm, pooled_hbm, idxv, tiles, pv,
                     (sem0, sem1), 192, 8, 96)

    @pl.when((wid >= 13) & (wid < 16))
    def _():
        pv[...] = jnp.zeros((16,), jnp.float32)
        pltpu.sync_copy(pv.at[pl.ds(0, 8)], pooled_hbm.at[pl.ds(wid * 8, 8)])


_sc_pool = pl.kernel(
    _sc_pool_body,
    out_type=jax.ShapeDtypeStruct((128,), jnp.float32),
    mesh=plsc.VectorSubcoreMesh(core_axis_name="c", subcore_axis_name="s"),
    compiler_params=pltpu.CompilerParams(
        needs_layout_passes=False, use_tc_tiling_on_sc=True),
    scratch_types=[
        pltpu.VMEM((16,), jnp.int32),
        pltpu.VMEM((2, NB, NT, 8, 128), jnp.float32),
        pltpu.VMEM((16,), jnp.float32),
        pltpu.SemaphoreType.DMA,
        pltpu.SemaphoreType.DMA,
    ],
)


def _tc_linear_body(pooled_ref, wt_ref, b_ref, out_ref):
    p = pooled_ref[:, :EMBED]
    y = lax.dot_general(p, wt_ref[...], (((1,), (0,)), ((), ())),
                        preferred_element_type=jnp.float32)
    out_ref[...] = y + b_ref[...]


def kernel(input, emb_table, W, b):
    idx = input.reshape(SEQ).astype(jnp.int32)
    embT = emb_table.T
    pooled = _sc_pool(idx, embT)
    wt = W.T
    return pl.pallas_call(
        _tc_linear_body,
        out_shape=jax.ShapeDtypeStruct((1, LABEL), jnp.float32),
    )(pooled.reshape(1, 128), wt, b.reshape(1, LABEL))
